# baseline (speedup 1.0000x reference)
import jax
import jax.numpy as jnp
from jax import lax
from jax.experimental import pallas as pl
from jax.experimental.pallas import tpu as pltpu
from jax.experimental.pallas import tpu_sc as plsc

NCORES = 2
NSUB = 16
LANES = 16
NPIX = 512 * 512
CHUNK = NPIX // NSUB
NCH = 24
BLK = 8
NBINS = 256
ROWS = 512 // NSUB


def _floorf(x):
    return x.astype(jnp.int32).astype(jnp.float32)


def _equalize_body(img, out, pix_a, pix_b, rm_a, rm_b, hist24, part,
                   histred, cum, lutall, lut_a, lut_b, hist_sh, lut_sh,
                   sem_ia, sem_ib, sem_ra, sem_rb, sem_oa, sem_ob,
                   sem_la, sem_lb):
    c = lax.axis_index("c")
    s = lax.axis_index("s")
    iota = lax.iota(jnp.int32, LANES)
    ones = jnp.ones((LANES,), jnp.float32)
    zeros = jnp.zeros((LANES,), jnp.float32)

    def in_slice(ch):
        return img.at[c * NCH + ch, pl.ds(s * ROWS, ROWS)]

    def out_slice(ch):
        return out.at[c * NCH + ch, pl.ds(s * ROWS, ROWS)]

    def scatter_chunk(pix, ch):
        chv = jnp.full((LANES,), ch, jnp.int32)

        @plsc.parallel_loop(0, CHUNK // LANES, 1, unroll=16)
        def _(i):
            v = plsc.bitcast(pix[i >> 5, pl.ds((i & 31) * LANES, LANES)],
                             jnp.int32)
            plsc.addupdate_scatter(hist24, [chv, v], ones)

    def gather_chunk(rm, lutrep):
        @plsc.parallel_loop(0, CHUNK // LANES, 1, unroll=16)
        def _(i):
            r = i >> 5
            sl = pl.ds((i & 31) * LANES, LANES)
            flat = (plsc.bitcast(rm[r, sl], jnp.int32) << 4) + iota
            rm[r, sl] = plsc.load_gather(lutrep, [flat >> 7, flat & 127])

    def make_lut(chv):
        pltpu.sync_copy(hist_sh.at[:, chv], part)

        @plsc.parallel_loop(0, NBINS // LANES, 1, unroll=2)
        def _(j):
            acc = part[0, pl.ds(j * LANES, LANES)]
            for r in range(1, NSUB):
                acc = acc + part[r, pl.ds(j * LANES, LANES)]
            histred[pl.ds(j * LANES, LANES)] = acc

        def cbody(j, carry):
            cacc, li = carry
            x = histred[pl.ds(j * LANES, LANES)]
            cs = plsc.cumsum(x) + cacc
            cum[pl.ds(j * LANES, LANES)] = cs
            gidx = iota + j * LANES
            ljm = jnp.max(jnp.where(x > 0.0, gidx, -1))
            return (jnp.max(cs), jnp.maximum(li, ljm))
        total, li = lax.fori_loop(
            0, NBINS // LANES, cbody, (jnp.float32(0.0), jnp.int32(-1)))

        def hbody(j, acc):
            x = histred[pl.ds(j * LANES, LANES)]
            gidx = iota + j * LANES
            return acc + jnp.sum(jnp.where(gidx == li, x, 0.0))
        hist_last = lax.fori_loop(0, NBINS // LANES, hbody, jnp.float32(0.0))

        num_v = jnp.full((LANES,), total - hist_last, jnp.float32)
        step = _floorf(num_v / 255.0)
        half = _floorf(step * 0.5)
        div = jnp.maximum(step, 1.0)
        ident = step <= 0.0
        chs = jnp.full((LANES,), chv, jnp.int32)

        def lbody(j, _):
            cs = cum[pl.ds(j * LANES, LANES)]
            val = jnp.clip(_floorf((cs + half) / div), 0.0, 255.0)
            gidx = iota + j * LANES
            val = jnp.where(ident, (gidx + 1).astype(jnp.float32), val)
            plsc.store_scatter(
                lutall, [chs, gidx + 1], val, mask=gidx < NBINS - 1)
            return 0
        lax.fori_loop(0, NBINS // LANES, lbody, 0)
        v0 = lutall[chv, pl.ds(0, LANES)]
        lutall[chv, pl.ds(0, LANES)] = jnp.where(iota == 0, 0.0, v0)

        def ibody(j, _):
            x = lutall[chv, pl.ds(j * LANES, LANES)]
            for k in range(LANES):
                val = jnp.full((LANES,), x[k], jnp.float32)
                lut_a[2 * j + k // 8, pl.ds((k & 7) * LANES, LANES)] = val
            return 0
        lax.fori_loop(0, NBINS // LANES, ibody, 0)
        pltpu.sync_copy(lut_a, lut_sh.at[chv])

    def fire_scatter(ch, pix, sem):
        pltpu.async_copy(in_slice(ch), pix, sem)

    def wait_scatter(ch, pix, sem):
        pltpu.make_async_copy(in_slice(ch), pix, sem).wait()

    def fire_remap(ch, rm, lut, semr, seml):
        pltpu.async_copy(in_slice(ch), rm, semr)
        pltpu.async_copy(lut_sh.at[ch], lut, seml)

    def wait_remap(ch, rm, lut, semr, seml):
        pltpu.make_async_copy(in_slice(ch), rm, semr).wait()
        pltpu.make_async_copy(lut_sh.at[ch], lut, seml).wait()

    def scatter_block(sc0):
        def body(j, _):
            ch = sc0 + 2 * j
            wait_scatter(ch, pix_a, sem_ia)
            scatter_chunk(pix_a, ch)

            @pl.when(ch + 2 < sc0 + BLK)
            def _():
                fire_scatter(ch + 2, pix_a, sem_ia)
            wait_scatter(ch + 1, pix_b, sem_ib)
            scatter_chunk(pix_b, ch + 1)

            @pl.when(ch + 3 < sc0 + BLK)
            def _():
                fire_scatter(ch + 3, pix_b, sem_ib)
            return 0
        lax.fori_loop(0, BLK // 2, body, 0)

    def remap_one(ch, rm, lut, semr, seml, semo, first):
        wait_remap(ch, rm, lut, semr, seml)
        if first:
            pass
        else:
            pltpu.make_async_copy(rm, out_slice(ch - 2), semo).wait()
        gather_chunk(rm, lut)
        pltpu.async_copy(rm, out_slice(ch), semo)

    def mix_block(sc0):
        rm0 = sc0 - BLK

        def body(j, _):
            ch = sc0 + 2 * j
            rch = rm0 + 2 * j
            wait_scatter(ch, pix_a, sem_ia)
            scatter_chunk(pix_a, ch)

            @pl.when(ch + 2 < sc0 + BLK)
            def _():
                fire_scatter(ch + 2, pix_a, sem_ia)
            remap_one(rch, rm_a, lut_a, sem_ra, sem_la, sem_oa,
                      first=False) if sc0 > BLK else None
            if sc0 == BLK:
                wait_remap(rch, rm_a, lut_a, sem_ra, sem_la)

                @pl.when(j > 0)
                def _():
                    pltpu.make_async_copy(
                        rm_a, out_slice(rch - 2), sem_oa).wait()
                gather_chunk(rm_a, lut_a)
                pltpu.async_copy(rm_a, out_slice(rch), sem_oa)

            @pl.when(rch + 2 < rm0 + BLK)
            def _():
                fire_remap(rch + 2, rm_a, lut_a, sem_ra, sem_la)

            wait_scatter(ch + 1, pix_b, sem_ib)
            scatter_chunk(pix_b, ch + 1)

            @pl.when(ch + 3 < sc0 + BLK)
            def _():
                fire_scatter(ch + 3, pix_b, sem_ib)
            if sc0 == BLK:
                wait_remap(rch + 1, rm_b, lut_b, sem_rb, sem_lb)

                @pl.when(j > 0)
                def _():
                    pltpu.make_async_copy(
                        rm_b, out_slice(rch - 1), sem_ob).wait()
                gather_chunk(rm_b, lut_b)
                pltpu.async_copy(rm_b, out_slice(rch + 1), sem_ob)
            else:
                remap_one(rch + 1, rm_b, lut_b, sem_rb, sem_lb, sem_ob,
                          first=False)

            @pl.when(rch + 3 < rm0 + BLK)
            def _():
                fire_remap(rch + 3, rm_b, lut_b, sem_rb, sem_lb)
            return 0
        lax.fori_loop(0, BLK // 2, body, 0)

    def remap_block(rm0):
        def body(j, _):
            rch = rm0 + 2 * j
            remap_one(rch, rm_a, lut_a, sem_ra, sem_la, sem_oa, first=False)

            @pl.when(rch + 2 < rm0 + BLK)
            def _():
                fire_remap(rch + 2, rm_a, lut_a, sem_ra, sem_la)
            remap_one(rch + 1, rm_b, lut_b, sem_rb, sem_lb, sem_ob,
                      first=False)

            @pl.when(rch + 3 < rm0 + BLK)
            def _():
                fire_remap(rch + 3, rm_b, lut_b, sem_rb, sem_lb)
            return 0
        lax.fori_loop(0, BLK // 2, body, 0)

    def lut_block(ch0):
        pltpu.sync_copy(hist24.at[pl.ds(ch0, BLK)],
                        hist_sh.at[s, pl.ds(ch0, BLK)])
        plsc.subcore_barrier()

        @pl.when(s < BLK)
        def _():
            make_lut(ch0 + s)
        plsc.subcore_barrier()

    @plsc.parallel_loop(0, NCH * NBINS // LANES, 1, unroll=4)
    def _(j):
        r = j >> 4
        col = (j & 15) * LANES
        hist24[r, pl.ds(col, LANES)] = zeros

    fire_scatter(0, pix_a, sem_ia)
    fire_scatter(1, pix_b, sem_ib)
    scatter_block(0)
    lut_block(0)
    fire_scatter(BLK, pix_a, sem_ia)
    fire_scatter(BLK + 1, pix_b, sem_ib)
    fire_remap(0, rm_a, lut_a, sem_ra, sem_la)
    fire_remap(1, rm_b, lut_b, sem_rb, sem_lb)
    mix_block(BLK)
    lut_block(BLK)
    fire_scatter(2 * BLK, pix_a, sem_ia)
    fire_scatter(2 * BLK + 1, pix_b, sem_ib)
    fire_remap(BLK, rm_a, lut_a, sem_ra, sem_la)
    fire_remap(BLK + 1, rm_b, lut_b, sem_rb, sem_lb)
    mix_block(2 * BLK)
    lut_block(2 * BLK)
    fire_remap(2 * BLK, rm_a, lut_a, sem_ra, sem_la)
    fire_remap(2 * BLK + 1, rm_b, lut_b, sem_rb, sem_lb)
    remap_block(2 * BLK)
    pltpu.make_async_copy(rm_a, out_slice(NCH - 2), sem_oa).wait()
    pltpu.make_async_copy(rm_b, out_slice(NCH - 1), sem_ob).wait()


@jax.jit
def kernel(image):
    B, C, H, W = image.shape
    flat = jax.lax.bitcast_convert_type(
        image.reshape(B * C, H, W), jnp.float32)

    mesh = plsc.VectorSubcoreMesh(
        core_axis_name="c", subcore_axis_name="s",
        num_cores=NCORES, num_subcores=NSUB)
    eq = pl.kernel(
        _equalize_body,
        out_type=jax.ShapeDtypeStruct((B * C, H, W), jnp.float32),
        mesh=mesh,
        compiler_params=pltpu.CompilerParams(
            use_tc_tiling_on_sc=True, needs_layout_passes=False),
        scratch_types=[
            pltpu.VMEM((ROWS, 512), jnp.float32),
            pltpu.VMEM((ROWS, 512), jnp.float32),
            pltpu.VMEM((ROWS, 512), jnp.float32),
            pltpu.VMEM((ROWS, 512), jnp.float32),
            pltpu.VMEM((NCH, NBINS), jnp.float32),
            pltpu.VMEM((NSUB, NBINS), jnp.float32),
            pltpu.VMEM((NBINS,), jnp.float32),
            pltpu.VMEM((NBINS,), jnp.float32),
            pltpu.VMEM((NCH, NBINS), jnp.float32),
            pltpu.VMEM((NBINS // 8, 128), jnp.float32),
            pltpu.VMEM((NBINS // 8, 128), jnp.float32),
            pltpu.VMEM_SHARED((NSUB, NCH, NBINS), jnp.float32),
            pltpu.VMEM_SHARED((NCH, NBINS // 8, 128), jnp.float32),
            pltpu.SemaphoreType.DMA,
            pltpu.SemaphoreType.DMA,
            pltpu.SemaphoreType.DMA,
            pltpu.SemaphoreType.DMA,
            pltpu.SemaphoreType.DMA,
            pltpu.SemaphoreType.DMA,
            pltpu.SemaphoreType.DMA,
            pltpu.SemaphoreType.DMA,
        ],
    )
    return eq(flat).reshape(B, C, H, W)


# speedup vs baseline: 1.2830x; 1.2830x over previous
import jax
import jax.numpy as jnp
from jax import lax
from jax.experimental import pallas as pl
from jax.experimental.pallas import tpu as pltpu
from jax.experimental.pallas import tpu_sc as plsc

NCORES = 2
NSUB = 16
LANES = 16
NPIX = 512 * 512
CHUNK = NPIX // NSUB
NCH = 24
NBINS = 256
ROWS = 512 // NSUB


def _floorf(x):
    return x.astype(jnp.int32).astype(jnp.float32)


def _equalize_body(img, out, pix_a, pix_b, out_a, out_b, hist24, part,
                   histred, cum, lutall, lut_a, lut_b, hist_sh, lut_sh,
                   sem_ia, sem_ib, sem_oa, sem_ob, sem_la, sem_lb):
    c = lax.axis_index("c")
    s = lax.axis_index("s")
    iota = lax.iota(jnp.int32, LANES)
    ones = jnp.ones((LANES,), jnp.float32)
    zeros = jnp.zeros((LANES,), jnp.float32)

    def in_slice(ch):
        return img.at[c * NCH + ch, pl.ds(s * ROWS, ROWS)]

    def out_slice(ch):
        return out.at[c * NCH + ch, pl.ds(s * ROWS, ROWS)]

    @plsc.parallel_loop(0, NCH * NBINS // LANES, 1, unroll=4)
    def _(j):
        r = j >> 4
        col = (j & 15) * LANES
        hist24[r, pl.ds(col, LANES)] = zeros

    def scatter_chunk(pix, ch):
        chv = jnp.full((LANES,), ch, jnp.int32)

        @plsc.parallel_loop(0, CHUNK // LANES, 1, unroll=16)
        def _(i):
            v = pix[i >> 5, pl.ds((i & 31) * LANES, LANES)]
            plsc.addupdate_scatter(hist24, [chv, v], ones)

    pltpu.async_copy(in_slice(0), pix_a, sem_ia)

    def p1_body(j, _):
        ch_a = 2 * j
        ch_b = 2 * j + 1
        pltpu.async_copy(in_slice(ch_b), pix_b, sem_ib)
        pltpu.make_async_copy(in_slice(ch_a), pix_a, sem_ia).wait()
        scatter_chunk(pix_a, ch_a)

        @pl.when(ch_a + 2 < NCH)
        def _():
            pltpu.async_copy(in_slice(ch_a + 2), pix_a, sem_ia)
        pltpu.make_async_copy(in_slice(ch_b), pix_b, sem_ib).wait()
        scatter_chunk(pix_b, ch_b)
        return 0
    lax.fori_loop(0, NCH // 2, p1_body, 0)

    pltpu.sync_copy(hist24, hist_sh.at[s])
    plsc.subcore_barrier()

    def make_lut(chv):
        pltpu.sync_copy(hist_sh.at[:, chv], part)

        @plsc.parallel_loop(0, NBINS // LANES, 1, unroll=2)
        def _(j):
            acc = part[0, pl.ds(j * LANES, LANES)]
            for r in range(1, NSUB):
                acc = acc + part[r, pl.ds(j * LANES, LANES)]
            histred[pl.ds(j * LANES, LANES)] = acc

        def cbody(j, carry):
            cacc, li = carry
            x = histred[pl.ds(j * LANES, LANES)]
            cs = plsc.cumsum(x) + cacc
            cum[pl.ds(j * LANES, LANES)] = cs
            gidx = iota + j * LANES
            ljm = jnp.max(jnp.where(x > 0.0, gidx, -1))
            return (jnp.max(cs), jnp.maximum(li, ljm))
        total, li = lax.fori_loop(
            0, NBINS // LANES, cbody, (jnp.float32(0.0), jnp.int32(-1)))

        def hbody(j, acc):
            x = histred[pl.ds(j * LANES, LANES)]
            gidx = iota + j * LANES
            return acc + jnp.sum(jnp.where(gidx == li, x, 0.0))
        hist_last = lax.fori_loop(0, NBINS // LANES, hbody, jnp.float32(0.0))

        num_v = jnp.full((LANES,), total - hist_last, jnp.float32)
        step = _floorf(num_v / 255.0)
        half = _floorf(step * 0.5)
        div = jnp.maximum(step, 1.0)
        ident = step <= 0.0
        chs = jnp.full((LANES,), chv, jnp.int32)

        def lbody(j, _):
            cs = cum[pl.ds(j * LANES, LANES)]
            val = jnp.clip(_floorf((cs + half) / div), 0.0, 255.0)
            gidx = iota + j * LANES
            val = jnp.where(ident, (gidx + 1).astype(jnp.float32), val)
            plsc.store_scatter(
                lutall, [chs, gidx + 1], val, mask=gidx < NBINS - 1)
            return 0
        lax.fori_loop(0, NBINS // LANES, lbody, 0)
        v0 = lutall[chv, pl.ds(0, LANES)]
        lutall[chv, pl.ds(0, LANES)] = jnp.where(iota == 0, 0.0, v0)

        def ibody(j, _):
            x = lutall[chv, pl.ds(j * LANES, LANES)]
            for k in range(LANES):
                val = jnp.full((LANES,), x[k], jnp.float32)
                lut_a[2 * j + k // 8, pl.ds((k & 7) * LANES, LANES)] = val
            return 0
        lax.fori_loop(0, NBINS // LANES, ibody, 0)
        pltpu.sync_copy(lut_a, lut_sh.at[chv])

    for rep in range(2):
        chx = s + NSUB * rep

        @pl.when(chx < NCH)
        def _(chx=chx):
            make_lut(chx)

    plsc.subcore_barrier()

    def gather_chunk(pix, lutrep, outb):
        @plsc.parallel_loop(0, CHUNK // LANES, 1, unroll=16)
        def _(i):
            r = i >> 5
            sl = pl.ds((i & 31) * LANES, LANES)
            flat = (pix[r, sl] << 4) + iota
            outb[r, sl] = plsc.load_gather(
                lutrep, [flat >> 7, flat & 127])

    pltpu.async_copy(in_slice(0), pix_a, sem_ia)
    pltpu.async_copy(lut_sh.at[0], lut_a, sem_la)

    def p2_body(j, _):
        ch_a = 2 * j
        ch_b = 2 * j + 1
        pltpu.async_copy(in_slice(ch_b), pix_b, sem_ib)
        pltpu.async_copy(lut_sh.at[ch_b], lut_b, sem_lb)
        pltpu.make_async_copy(in_slice(ch_a), pix_a, sem_ia).wait()
        pltpu.make_async_copy(lut_sh.at[ch_a], lut_a, sem_la).wait()

        @pl.when(j > 0)
        def _():
            pltpu.make_async_copy(out_a, out_slice(ch_a - 2), sem_oa).wait()
        gather_chunk(pix_a, lut_a, out_a)
        pltpu.async_copy(out_a, out_slice(ch_a), sem_oa)

        @pl.when(ch_a + 2 < NCH)
        def _():
            pltpu.async_copy(in_slice(ch_a + 2), pix_a, sem_ia)
            pltpu.async_copy(lut_sh.at[ch_a + 2], lut_a, sem_la)
        pltpu.make_async_copy(in_slice(ch_b), pix_b, sem_ib).wait()
        pltpu.make_async_copy(lut_sh.at[ch_b], lut_b, sem_lb).wait()

        @pl.when(j > 0)
        def _():
            pltpu.make_async_copy(out_b, out_slice(ch_b - 2), sem_ob).wait()
        gather_chunk(pix_b, lut_b, out_b)
        pltpu.async_copy(out_b, out_slice(ch_b), sem_ob)
        return 0
    lax.fori_loop(0, NCH // 2, p2_body, 0)

    pltpu.make_async_copy(out_a, out_slice(NCH - 2), sem_oa).wait()
    pltpu.make_async_copy(out_b, out_slice(NCH - 1), sem_ob).wait()


@jax.jit
def kernel(image):
    B, C, H, W = image.shape
    flat = image.reshape(B * C, H, W)

    mesh = plsc.VectorSubcoreMesh(
        core_axis_name="c", subcore_axis_name="s",
        num_cores=NCORES, num_subcores=NSUB)
    eq = pl.kernel(
        _equalize_body,
        out_type=jax.ShapeDtypeStruct((B * C, H, W), jnp.float32),
        mesh=mesh,
        compiler_params=pltpu.CompilerParams(
            use_tc_tiling_on_sc=True, needs_layout_passes=False),
        scratch_types=[
            pltpu.VMEM((ROWS, 512), jnp.int32),
            pltpu.VMEM((ROWS, 512), jnp.int32),
            pltpu.VMEM((ROWS, 512), jnp.float32),
            pltpu.VMEM((ROWS, 512), jnp.float32),
            pltpu.VMEM((NCH, NBINS), jnp.float32),
            pltpu.VMEM((NSUB, NBINS), jnp.float32),
            pltpu.VMEM((NBINS,), jnp.float32),
            pltpu.VMEM((NBINS,), jnp.float32),
            pltpu.VMEM((NCH, NBINS), jnp.float32),
            pltpu.VMEM((NBINS // 8, 128), jnp.float32),
            pltpu.VMEM((NBINS // 8, 128), jnp.float32),
            pltpu.VMEM_SHARED((NSUB, NCH, NBINS), jnp.float32),
            pltpu.VMEM_SHARED((NCH, NBINS // 8, 128), jnp.float32),
            pltpu.SemaphoreType.DMA,
            pltpu.SemaphoreType.DMA,
            pltpu.SemaphoreType.DMA,
            pltpu.SemaphoreType.DMA,
            pltpu.SemaphoreType.DMA,
            pltpu.SemaphoreType.DMA,
        ],
    )
    return eq(flat).reshape(B, C, H, W)
